# Initial kernel scaffold; baseline (speedup 1.0000x reference)
import functools

import jax
import jax.numpy as jnp
import numpy as np
from jax import lax
from jax.experimental import pallas as pl
from jax.experimental.pallas import tpu as pltpu
from jax.experimental.pallas import tpu_sc as plsc

B = 16384
F = 13
D = 128
NF = F + 1

NC = 2
NS = 16
NW = NC * NS
PER_W = B * F // NW
CHUNK = 128
NCH = PER_W // CHUNK


def _gather_body(idx_hbm, table_hbm, out_hbm, idx_v, buf0, buf1, sem):
  cid = lax.axis_index("c")
  sid = lax.axis_index("s")
  wid = sid * NC + cid
  rowbase = wid * NCH
  outbase = wid * PER_W
  pltpu.sync_copy(idx_hbm.at[pl.ds(rowbase, NCH)], idx_v)
  bufs = (buf0, buf1)
  pltpu.async_copy(table_hbm.at[idx_v.at[0]], bufs[0], sem.at[0])

  @pl.loop(0, NCH, step=2)
  def _(c):
    for b in range(2):
      ci = c + b
      pltpu.make_async_copy(table_hbm.at[idx_v.at[ci]], bufs[b], sem.at[b]).wait()
      nxt = lax.rem(ci + 1, NCH)
      pltpu.async_copy(table_hbm.at[idx_v.at[nxt]], bufs[1 - b], sem.at[1 - b])
      pltpu.sync_copy(bufs[b], out_hbm.at[pl.ds(outbase + ci * CHUNK, CHUNK)])

  pltpu.make_async_copy(table_hbm.at[idx_v.at[0]], bufs[0], sem.at[0]).wait()


_sc_gather = pl.kernel(
    _gather_body,
    out_type=jax.ShapeDtypeStruct((B * F, D), jnp.float32),
    mesh=plsc.VectorSubcoreMesh(
        core_axis_name="c", subcore_axis_name="s", num_cores=NC, num_subcores=NS
    ),
    scratch_types=[
        pltpu.VMEM((NCH, CHUNK), jnp.int32),
        pltpu.VMEM((CHUNK, D), jnp.float32),
        pltpu.VMEM((CHUNK, D), jnp.float32),
        pltpu.SemaphoreType.DMA((2,)),
    ],
)


BB = 512
NBLK = B // BB


def _dense_body(df_ref, g_ref, dw0, db0, dw1, db1, dw2, db2,
                ow0a, ow0b, ob0, ow1, ob1, ow2, ob2, ow3, ob3, ow4, ob4,
                out_ref):
  f32 = jnp.float32
  x = df_ref[:]
  x = jnp.maximum(jnp.dot(x, dw0[:], preferred_element_type=f32) + db0[:], 0.0)
  x = jnp.maximum(jnp.dot(x, dw1[:], preferred_element_type=f32) + db1[:], 0.0)
  ed = jnp.maximum(jnp.dot(x, dw2[:], preferred_element_type=f32) + db2[:], 0.0)
  g = g_ref[:]
  feats = [ed] + [g[:, f * D:(f + 1) * D] for f in range(F)]
  cols = []
  for f in range(NF):
    for h in range(f + 1, NF):
      cols.append(jnp.sum(feats[f] * feats[h], axis=1, keepdims=True))
  flat = jnp.concatenate(cols, axis=1)
  y = (jnp.dot(ed, ow0a[:], preferred_element_type=f32)
       + jnp.dot(flat, ow0b[:], preferred_element_type=f32) + ob0[:])
  y = jnp.maximum(y, 0.0)
  y = jnp.maximum(jnp.dot(y, ow1[:], preferred_element_type=f32) + ob1[:], 0.0)
  y = jnp.maximum(jnp.dot(y, ow2[:], preferred_element_type=f32) + ob2[:], 0.0)
  y = jnp.maximum(jnp.dot(y, ow3[:], preferred_element_type=f32) + ob3[:], 0.0)
  out_ref[:] = jnp.dot(y, ow4[:], preferred_element_type=f32) + ob4[:]


def _full(shape):
  return pl.BlockSpec(shape, lambda i: (0, 0))


def _dense_call(df, g2, dw0, db0, dw1, db1, dw2, db2,
                ow0a, ow0b, ob0, ow1, ob1, ow2, ob2, ow3, ob3, ow4, ob4):
  in_specs = [
      pl.BlockSpec((BB, 13), lambda i: (i, 0)),
      pl.BlockSpec((BB, F * D), lambda i: (i, 0)),
  ]
  for w in (dw0, db0, dw1, db1, dw2, db2,
            ow0a, ow0b, ob0, ow1, ob1, ow2, ob2, ow3, ob3, ow4, ob4):
    in_specs.append(_full(w.shape))
  return pl.pallas_call(
      _dense_body,
      grid=(NBLK,),
      in_specs=in_specs,
      out_specs=pl.BlockSpec((BB, 1), lambda i: (i, 0)),
      out_shape=jax.ShapeDtypeStruct((B, 1), jnp.float32),
      compiler_params=pltpu.CompilerParams(
          dimension_semantics=("arbitrary",),
      ),
  )(df, g2, dw0, db0, dw1, db1, dw2, db2,
    ow0a, ow0b, ob0, ow1, ob1, ow2, ob2, ow3, ob3, ow4, ob4)


@jax.jit
def kernel(dense_features, sparse_values, sparse_offsets, emb_table,
           dw0, db0, dw1, db1, dw2, db2,
           ow0, ob0, ow1, ob1, ow2, ob2, ow3, ob3, ow4, ob4):
  del sparse_offsets
  idx2d = sparse_values.reshape(NW * NCH, CHUNK)
  gathered = _sc_gather(idx2d, emb_table)
  g2 = gathered.reshape(B, F * D)
  ow0a = ow0[:D]
  ow0b = ow0[D:]
  r = lambda b: b.reshape(1, -1)
  return _dense_call(
      dense_features, g2,
      dw0, r(db0), dw1, r(db1), dw2, r(db2),
      ow0a, ow0b, r(ob0), ow1, r(ob1), ow2, r(ob2), ow3, r(ob3), ow4, r(ob4))


# speedup vs baseline: 8.8847x; 8.8847x over previous
import functools

import jax
import jax.numpy as jnp
import numpy as np
from jax import lax
from jax.experimental import pallas as pl
from jax.experimental.pallas import tpu as pltpu
from jax.experimental.pallas import tpu_sc as plsc

B = 16384
F = 13
D = 128
NF = F + 1

NC = 2
NS = 16
NW = NC * NS
PER_W = B * F // NW
CHUNK = 128
NCH = PER_W // CHUNK


def _gather_body(idx_hbm, table_hbm, out_hbm, idx_v, buf0, buf1, sem):
  cid = lax.axis_index("c")
  sid = lax.axis_index("s")
  wid = sid * NC + cid
  outbase = wid * PER_W
  pltpu.sync_copy(idx_hbm.at[wid], idx_v)
  bufs = (buf0, buf1)
  pltpu.async_copy(table_hbm.at[idx_v.at[0]], bufs[0], sem.at[0])

  @pl.loop(0, NCH, step=2)
  def _(c):
    for b in range(2):
      ci = c + b
      pltpu.make_async_copy(table_hbm.at[idx_v.at[ci]], bufs[b], sem.at[b]).wait()
      nxt = lax.rem(ci + 1, NCH)
      pltpu.async_copy(table_hbm.at[idx_v.at[nxt]], bufs[1 - b], sem.at[1 - b])
      pltpu.sync_copy(bufs[b], out_hbm.at[pl.ds(outbase + ci * CHUNK, CHUNK)])

  pltpu.make_async_copy(table_hbm.at[idx_v.at[0]], bufs[0], sem.at[0]).wait()


@functools.cache
def _sc_gather():
  return pl.kernel(
      _gather_body,
      out_type=jax.ShapeDtypeStruct((B * F, D), jnp.float32),
      mesh=plsc.VectorSubcoreMesh(
          core_axis_name="c", subcore_axis_name="s", num_cores=NC,
          num_subcores=NS,
      ),
      scratch_types=[
          pltpu.VMEM((NCH, CHUNK), jnp.int32),
          pltpu.VMEM((CHUNK, D), jnp.float32),
          pltpu.VMEM((CHUNK, D), jnp.float32),
          pltpu.SemaphoreType.DMA((2,)),
      ],
  )


BB = 512
NBLK = B // BB


def _dense_body(df_ref, g_ref, dw0, db0, dw1, db1, dw2, db2,
                ow0a, ow0b, ob0, ow1, ob1, ow2, ob2, ow3, ob3, ow4, ob4,
                out_ref):
  f32 = jnp.float32
  x = df_ref[:]
  x = jnp.maximum(jnp.dot(x, dw0[:], preferred_element_type=f32) + db0[:], 0.0)
  x = jnp.maximum(jnp.dot(x, dw1[:], preferred_element_type=f32) + db1[:], 0.0)
  ed = jnp.maximum(jnp.dot(x, dw2[:], preferred_element_type=f32) + db2[:], 0.0)
  g = g_ref[:]
  feats = [ed] + [g[:, f * D:(f + 1) * D] for f in range(F)]
  cols = []
  for f in range(NF):
    for h in range(f + 1, NF):
      cols.append(jnp.sum(feats[f] * feats[h], axis=1, keepdims=True))
  flat = jnp.concatenate(cols, axis=1)
  y = (jnp.dot(ed, ow0a[:], preferred_element_type=f32)
       + jnp.dot(flat, ow0b[:], preferred_element_type=f32) + ob0[:])
  y = jnp.maximum(y, 0.0)
  y = jnp.maximum(jnp.dot(y, ow1[:], preferred_element_type=f32) + ob1[:], 0.0)
  y = jnp.maximum(jnp.dot(y, ow2[:], preferred_element_type=f32) + ob2[:], 0.0)
  y = jnp.maximum(jnp.dot(y, ow3[:], preferred_element_type=f32) + ob3[:], 0.0)
  out_ref[:] = jnp.dot(y, ow4[:], preferred_element_type=f32) + ob4[:]


def _full(shape):
  return pl.BlockSpec(shape, lambda i: (0, 0))


def _dense_call(df, g2, dw0, db0, dw1, db1, dw2, db2,
                ow0a, ow0b, ob0, ow1, ob1, ow2, ob2, ow3, ob3, ow4, ob4):
  in_specs = [
      pl.BlockSpec((BB, 13), lambda i: (i, 0)),
      pl.BlockSpec((BB, F * D), lambda i: (i, 0)),
  ]
  for w in (dw0, db0, dw1, db1, dw2, db2,
            ow0a, ow0b, ob0, ow1, ob1, ow2, ob2, ow3, ob3, ow4, ob4):
    in_specs.append(_full(w.shape))
  return pl.pallas_call(
      _dense_body,
      grid=(NBLK,),
      in_specs=in_specs,
      out_specs=pl.BlockSpec((BB, 1), lambda i: (i, 0)),
      out_shape=jax.ShapeDtypeStruct((B, 1), jnp.float32),
      compiler_params=pltpu.CompilerParams(
          dimension_semantics=("arbitrary",),
      ),
  )(df, g2, dw0, db0, dw1, db1, dw2, db2,
    ow0a, ow0b, ob0, ow1, ob1, ow2, ob2, ow3, ob3, ow4, ob4)


@jax.jit
def kernel(dense_features, sparse_values, sparse_offsets, emb_table,
           dw0, db0, dw1, db1, dw2, db2,
           ow0, ob0, ow1, ob1, ow2, ob2, ow3, ob3, ow4, ob4):
  del sparse_offsets
  idx2d = sparse_values.reshape(NW, NCH, CHUNK)
  gathered = _sc_gather()(idx2d, emb_table)
  g2 = gathered.reshape(B, F * D)
  ow0a = ow0[:D]
  ow0b = ow0[D:]
  r = lambda b: b.reshape(1, -1)
  return _dense_call(
      dense_features, g2,
      dw0, r(db0), dw1, r(db1), dw2, r(db2),
      ow0a, ow0b, r(ob0), ow1, r(ob1), ow2, r(ob2), ow3, r(ob3), ow4, r(ob4))


# speedup vs baseline: 9.9167x; 1.1162x over previous
import functools

import jax
import jax.numpy as jnp
import numpy as np
from jax import lax
from jax.experimental import pallas as pl
from jax.experimental.pallas import tpu as pltpu
from jax.experimental.pallas import tpu_sc as plsc

B = 16384
F = 13
D = 128
NF = F + 1

NC = 2
NS = 16
NW = NC * NS
PER_W = B * F // NW
CHUNK = 128
NCH = PER_W // CHUNK


def _gather_body(idx_hbm, table_hbm, out_hbm, idx_v, buf0, buf1, sem):
  cid = lax.axis_index("c")
  sid = lax.axis_index("s")
  wid = sid * NC + cid
  outbase = wid * PER_W
  pltpu.sync_copy(idx_hbm.at[wid], idx_v)
  bufs = (buf0, buf1)
  pltpu.async_copy(table_hbm.at[idx_v.at[0]], bufs[0], sem.at[0])

  @pl.loop(0, NCH, step=2)
  def _(c):
    for b in range(2):
      ci = c + b
      pltpu.make_async_copy(table_hbm.at[idx_v.at[ci]], bufs[b], sem.at[b]).wait()
      nxt = lax.rem(ci + 1, NCH)
      pltpu.async_copy(table_hbm.at[idx_v.at[nxt]], bufs[1 - b], sem.at[1 - b])
      pltpu.sync_copy(bufs[b], out_hbm.at[pl.ds(outbase + ci * CHUNK, CHUNK)])

  pltpu.make_async_copy(table_hbm.at[idx_v.at[0]], bufs[0], sem.at[0]).wait()


@functools.cache
def _sc_gather():
  return pl.kernel(
      _gather_body,
      out_type=jax.ShapeDtypeStruct((B * F, D), jnp.float32),
      mesh=plsc.VectorSubcoreMesh(
          core_axis_name="c", subcore_axis_name="s", num_cores=NC,
          num_subcores=NS,
      ),
      scratch_types=[
          pltpu.VMEM((NCH, CHUNK), jnp.int32),
          pltpu.VMEM((CHUNK, D), jnp.float32),
          pltpu.VMEM((CHUNK, D), jnp.float32),
          pltpu.SemaphoreType.DMA((2,)),
      ],
  )


BB = 512
NBLK = B // BB


def _dense_body(df_ref, g_ref, dw0, db0, dw1, db1, dw2, db2,
                ow0a, ow0b, ob0, ow1, ob1, ow2, ob2, ow3, ob3, ow4, ob4,
                out_ref, ed_s, flat_s):
  i = pl.program_id(0)
  par = lax.rem(i, 2)
  oth = 1 - par
  f32 = jnp.float32

  edp = ed_s[oth]
  flatp = flat_s[oth]
  y = (jnp.dot(edp, ow0a[:], preferred_element_type=f32)
       + jnp.dot(flatp, ow0b[:], preferred_element_type=f32) + ob0[:])
  y = jnp.maximum(y, 0.0)
  y = jnp.maximum(jnp.dot(y, ow1[:], preferred_element_type=f32) + ob1[:], 0.0)
  y = jnp.maximum(jnp.dot(y, ow2[:], preferred_element_type=f32) + ob2[:], 0.0)
  y = jnp.maximum(jnp.dot(y, ow3[:], preferred_element_type=f32) + ob3[:], 0.0)
  out_ref[:] = jnp.dot(y, ow4[:], preferred_element_type=f32) + ob4[:]

  x = df_ref[:]
  x = jnp.maximum(jnp.dot(x, dw0[:], preferred_element_type=f32) + db0[:], 0.0)
  x = jnp.maximum(jnp.dot(x, dw1[:], preferred_element_type=f32) + db1[:], 0.0)
  ed = jnp.maximum(jnp.dot(x, dw2[:], preferred_element_type=f32) + db2[:], 0.0)
  g = g_ref[:]
  feats = [ed] + [g[:, f * D:(f + 1) * D] for f in range(F)]
  cols = []
  for f in range(NF):
    for h in range(f + 1, NF):
      cols.append(jnp.sum(feats[f] * feats[h], axis=1, keepdims=True))
  flat = jnp.concatenate(cols, axis=1)
  ed_s[par] = ed
  flat_s[par] = flat


def _full(shape):
  return pl.BlockSpec(shape, lambda i: (0, 0))


def _dense_call(df, g2, dw0, db0, dw1, db1, dw2, db2,
                ow0a, ow0b, ob0, ow1, ob1, ow2, ob2, ow3, ob3, ow4, ob4):
  clamp = lambda i: (jnp.minimum(i, NBLK - 1), 0)
  lag = lambda i: (jnp.maximum(i - 1, 0), 0)
  in_specs = [
      pl.BlockSpec((BB, 13), clamp),
      pl.BlockSpec((BB, F * D), clamp),
  ]
  for w in (dw0, db0, dw1, db1, dw2, db2,
            ow0a, ow0b, ob0, ow1, ob1, ow2, ob2, ow3, ob3, ow4, ob4):
    in_specs.append(_full(w.shape))
  return pl.pallas_call(
      _dense_body,
      grid=(NBLK + 1,),
      in_specs=in_specs,
      out_specs=pl.BlockSpec((BB, 1), lag),
      out_shape=jax.ShapeDtypeStruct((B, 1), jnp.float32),
      scratch_shapes=[
          pltpu.VMEM((2, BB, D), jnp.float32),
          pltpu.VMEM((2, BB, 91), jnp.float32),
      ],
      compiler_params=pltpu.CompilerParams(
          dimension_semantics=("arbitrary",),
      ),
  )(df, g2, dw0, db0, dw1, db1, dw2, db2,
    ow0a, ow0b, ob0, ow1, ob1, ow2, ob2, ow3, ob3, ow4, ob4)


@jax.jit
def kernel(dense_features, sparse_values, sparse_offsets, emb_table,
           dw0, db0, dw1, db1, dw2, db2,
           ow0, ob0, ow1, ob1, ow2, ob2, ow3, ob3, ow4, ob4):
  del sparse_offsets
  idx2d = sparse_values.reshape(NW, NCH, CHUNK)
  gathered = _sc_gather()(idx2d, emb_table)
  g2 = gathered.reshape(B, F * D)
  ow0a = ow0[:D]
  ow0b = ow0[D:]
  r = lambda b: b.reshape(1, -1)
  return _dense_call(
      dense_features, g2,
      dw0, r(db0), dw1, r(db1), dw2, r(db2),
      ow0a, ow0b, r(ob0), ow1, r(ob1), ow2, r(ob2), ow3, r(ob3), ow4, r(ob4))


# speedup vs baseline: 10.1300x; 1.0215x over previous
import functools

import jax
import jax.numpy as jnp
import numpy as np
from jax import lax
from jax.experimental import pallas as pl
from jax.experimental.pallas import tpu as pltpu
from jax.experimental.pallas import tpu_sc as plsc

B = 16384
F = 13
D = 128
NF = F + 1

NC = 2
NS = 16
NW = NC * NS
CHUNK = 128
SPLIT = 2


def _make_gather_body(nch):
  per_w = nch * CHUNK

  def _gather_body(idx_hbm, table_hbm, out_hbm, idx_v, buf0, buf1, sem):
    cid = lax.axis_index("c")
    sid = lax.axis_index("s")
    wid = sid * NC + cid
    outbase = wid * per_w
    pltpu.sync_copy(idx_hbm.at[wid], idx_v)
    bufs = (buf0, buf1)
    pltpu.async_copy(table_hbm.at[idx_v.at[0]], bufs[0], sem.at[0])

    @pl.loop(0, nch, step=2)
    def _(c):
      for b in range(2):
        ci = c + b
        pltpu.make_async_copy(table_hbm.at[idx_v.at[ci]], bufs[b], sem.at[b]).wait()
        nxt = lax.rem(ci + 1, nch)
        pltpu.async_copy(table_hbm.at[idx_v.at[nxt]], bufs[1 - b], sem.at[1 - b])
        pltpu.sync_copy(bufs[b], out_hbm.at[pl.ds(outbase + ci * CHUNK, CHUNK)])

    pltpu.make_async_copy(table_hbm.at[idx_v.at[0]], bufs[0], sem.at[0]).wait()

  return _gather_body


@functools.cache
def _sc_gather(nch):
  return pl.kernel(
      _make_gather_body(nch),
      out_type=jax.ShapeDtypeStruct((NW * nch * CHUNK, D), jnp.float32),
      mesh=plsc.VectorSubcoreMesh(
          core_axis_name="c", subcore_axis_name="s", num_cores=NC,
          num_subcores=NS,
      ),
      scratch_types=[
          pltpu.VMEM((nch, CHUNK), jnp.int32),
          pltpu.VMEM((CHUNK, D), jnp.float32),
          pltpu.VMEM((CHUNK, D), jnp.float32),
          pltpu.SemaphoreType.DMA((2,)),
      ],
  )


BB = 512
NBLK = B // BB


def _dense_body(df_ref, g_ref, dw0, db0, dw1, db1, dw2, db2,
                ow0a, ow0b, ob0, ow1, ob1, ow2, ob2, ow3, ob3, ow4, ob4,
                out_ref, ed_s, flat_s):
  i = pl.program_id(0)
  par = lax.rem(i, 2)
  oth = 1 - par
  f32 = jnp.float32

  edp = ed_s[oth]
  flatp = flat_s[oth]
  y = (jnp.dot(edp, ow0a[:], preferred_element_type=f32)
       + jnp.dot(flatp, ow0b[:], preferred_element_type=f32) + ob0[:])
  y = jnp.maximum(y, 0.0)
  y = jnp.maximum(jnp.dot(y, ow1[:], preferred_element_type=f32) + ob1[:], 0.0)
  y = jnp.maximum(jnp.dot(y, ow2[:], preferred_element_type=f32) + ob2[:], 0.0)
  y = jnp.maximum(jnp.dot(y, ow3[:], preferred_element_type=f32) + ob3[:], 0.0)
  out_ref[:] = jnp.dot(y, ow4[:], preferred_element_type=f32) + ob4[:]

  x = df_ref[:]
  x = jnp.maximum(jnp.dot(x, dw0[:], preferred_element_type=f32) + db0[:], 0.0)
  x = jnp.maximum(jnp.dot(x, dw1[:], preferred_element_type=f32) + db1[:], 0.0)
  ed = jnp.maximum(jnp.dot(x, dw2[:], preferred_element_type=f32) + db2[:], 0.0)
  g = g_ref[:]
  feats = [ed] + [g[:, f * D:(f + 1) * D] for f in range(F)]
  cols = []
  for f in range(NF):
    for h in range(f + 1, NF):
      cols.append(jnp.sum(feats[f] * feats[h], axis=1, keepdims=True))
  flat = jnp.concatenate(cols, axis=1)
  ed_s[par] = ed
  flat_s[par] = flat


def _full(shape):
  return pl.BlockSpec(shape, lambda i: (0, 0))


def _dense_call(df, g2, dw0, db0, dw1, db1, dw2, db2,
                ow0a, ow0b, ob0, ow1, ob1, ow2, ob2, ow3, ob3, ow4, ob4):
  nb = df.shape[0] // BB
  clamp = lambda i: (jnp.minimum(i, nb - 1), 0)
  lag = lambda i: (jnp.maximum(i - 1, 0), 0)
  in_specs = [
      pl.BlockSpec((BB, 13), clamp),
      pl.BlockSpec((BB, F * D), clamp),
  ]
  for w in (dw0, db0, dw1, db1, dw2, db2,
            ow0a, ow0b, ob0, ow1, ob1, ow2, ob2, ow3, ob3, ow4, ob4):
    in_specs.append(_full(w.shape))
  return pl.pallas_call(
      _dense_body,
      grid=(nb + 1,),
      in_specs=in_specs,
      out_specs=pl.BlockSpec((BB, 1), lag),
      out_shape=jax.ShapeDtypeStruct((df.shape[0], 1), jnp.float32),
      scratch_shapes=[
          pltpu.VMEM((2, BB, D), jnp.float32),
          pltpu.VMEM((2, BB, 91), jnp.float32),
      ],
      compiler_params=pltpu.CompilerParams(
          dimension_semantics=("arbitrary",),
      ),
  )(df, g2, dw0, db0, dw1, db1, dw2, db2,
    ow0a, ow0b, ob0, ow1, ob1, ow2, ob2, ow3, ob3, ow4, ob4)


@jax.jit
def kernel(dense_features, sparse_values, sparse_offsets, emb_table,
           dw0, db0, dw1, db1, dw2, db2,
           ow0, ob0, ow1, ob1, ow2, ob2, ow3, ob3, ow4, ob4):
  del sparse_offsets
  ow0a = ow0[:D]
  ow0b = ow0[D:]
  r = lambda b: b.reshape(1, -1)
  h = B // SPLIT
  nch = (h * F) // (NW * CHUNK)
  outs = []
  for k in range(SPLIT):
    idx3 = sparse_values[k * h * F:(k + 1) * h * F].reshape(NW, nch, CHUNK)
    gathered = _sc_gather(nch)(idx3, emb_table)
    g2 = gathered.reshape(h, F * D)
    outs.append(_dense_call(
        dense_features[k * h:(k + 1) * h], g2,
        dw0, r(db0), dw1, r(db1), dw2, r(db2),
        ow0a, ow0b, r(ob0), ow1, r(ob1), ow2, r(ob2), ow3, r(ob3),
        ow4, r(ob4)))
  return jnp.concatenate(outs, axis=0)


# speedup vs baseline: 13.5426x; 1.3369x over previous
import functools

import jax
import jax.numpy as jnp
import numpy as np
from jax import lax
from jax.experimental import pallas as pl
from jax.experimental.pallas import tpu as pltpu
from jax.experimental.pallas import tpu_sc as plsc

B = 16384
F = 13
D = 128
NF = F + 1

NC = 2
NS = 16
NW = NC * NS
CHUNK = 128
SPLIT = 2


def _make_gather_body(nch):
  per_w = nch * CHUNK

  def _gather_body(idx_hbm, table_hbm, out_hbm, idx_v, buf0, buf1, sem):
    cid = lax.axis_index("c")
    sid = lax.axis_index("s")
    wid = sid * NC + cid
    outbase = wid * per_w
    pltpu.sync_copy(idx_hbm.at[wid], idx_v)
    bufs = (buf0, buf1)
    pltpu.async_copy(table_hbm.at[idx_v.at[0]], bufs[0], sem.at[0])

    @pl.loop(0, nch, step=2)
    def _(c):
      for b in range(2):
        ci = c + b
        pltpu.make_async_copy(table_hbm.at[idx_v.at[ci]], bufs[b], sem.at[b]).wait()
        nxt = lax.rem(ci + 1, nch)
        pltpu.async_copy(table_hbm.at[idx_v.at[nxt]], bufs[1 - b], sem.at[1 - b])
        pltpu.sync_copy(bufs[b], out_hbm.at[pl.ds(outbase + ci * CHUNK, CHUNK)])

    pltpu.make_async_copy(table_hbm.at[idx_v.at[0]], bufs[0], sem.at[0]).wait()

  return _gather_body


@functools.cache
def _sc_gather(nch):
  return pl.kernel(
      _make_gather_body(nch),
      out_type=jax.ShapeDtypeStruct((NW * nch * CHUNK, D), jnp.float32),
      mesh=plsc.VectorSubcoreMesh(
          core_axis_name="c", subcore_axis_name="s", num_cores=NC,
          num_subcores=NS,
      ),
      scratch_types=[
          pltpu.VMEM((nch, CHUNK), jnp.int32),
          pltpu.VMEM((CHUNK, D), jnp.float32),
          pltpu.VMEM((CHUNK, D), jnp.float32),
          pltpu.SemaphoreType.DMA((2,)),
      ],
  )


BB = 512
NBLK = B // BB


def _dense_body(df_ref, g_ref, dw0, db0, dw1, db1, dw2, db2,
                ow0a, ow0b, ob0, ow1, ob1, ow2, ob2, ow3, ob3, ow4, ob4,
                out_ref, ed_s, flat_s):
  i = pl.program_id(0)
  par = lax.rem(i, 2)
  oth = 1 - par
  f32 = jnp.float32

  edp = ed_s[oth]
  flatp = flat_s[oth]
  y = (jnp.dot(edp, ow0a[:], preferred_element_type=f32)
       + jnp.dot(flatp, ow0b[:], preferred_element_type=f32) + ob0[:])
  y = jnp.maximum(y, 0.0)
  y = jnp.maximum(jnp.dot(y, ow1[:], preferred_element_type=f32) + ob1[:], 0.0)
  y = jnp.maximum(jnp.dot(y, ow2[:], preferred_element_type=f32) + ob2[:], 0.0)
  y = jnp.maximum(jnp.dot(y, ow3[:], preferred_element_type=f32) + ob3[:], 0.0)
  out_ref[:] = jnp.dot(y, ow4[:], preferred_element_type=f32) + ob4[:]

  x = df_ref[:]
  x = jnp.maximum(jnp.dot(x, dw0[:], preferred_element_type=f32) + db0[:], 0.0)
  x = jnp.maximum(jnp.dot(x, dw1[:], preferred_element_type=f32) + db1[:], 0.0)
  ed = jnp.maximum(jnp.dot(x, dw2[:], preferred_element_type=f32) + db2[:], 0.0)
  feats = [ed] + [g_ref[f] for f in range(F)]
  cols = []
  for f in range(NF):
    for h in range(f + 1, NF):
      cols.append(jnp.sum(feats[f] * feats[h], axis=1, keepdims=True))
  flat = jnp.concatenate(cols, axis=1)
  ed_s[par] = ed
  flat_s[par] = flat


def _full(shape):
  return pl.BlockSpec(shape, lambda i: (0, 0))


def _dense_call(df, g2, dw0, db0, dw1, db1, dw2, db2,
                ow0a, ow0b, ob0, ow1, ob1, ow2, ob2, ow3, ob3, ow4, ob4):
  nb = df.shape[0] // BB
  clamp = lambda i: (jnp.minimum(i, nb - 1), 0)
  lag = lambda i: (jnp.maximum(i - 1, 0), 0)
  in_specs = [
      pl.BlockSpec((BB, 13), clamp),
      pl.BlockSpec((F, BB, D), lambda i: (0, jnp.minimum(i, nb - 1), 0)),
  ]
  for w in (dw0, db0, dw1, db1, dw2, db2,
            ow0a, ow0b, ob0, ow1, ob1, ow2, ob2, ow3, ob3, ow4, ob4):
    in_specs.append(_full(w.shape))
  return pl.pallas_call(
      _dense_body,
      grid=(nb + 1,),
      in_specs=in_specs,
      out_specs=pl.BlockSpec((BB, 1), lag),
      out_shape=jax.ShapeDtypeStruct((df.shape[0], 1), jnp.float32),
      scratch_shapes=[
          pltpu.VMEM((2, BB, D), jnp.float32),
          pltpu.VMEM((2, BB, 91), jnp.float32),
      ],
      compiler_params=pltpu.CompilerParams(
          dimension_semantics=("arbitrary",),
      ),
  )(df, g2, dw0, db0, dw1, db1, dw2, db2,
    ow0a, ow0b, ob0, ow1, ob1, ow2, ob2, ow3, ob3, ow4, ob4)


@jax.jit
def kernel(dense_features, sparse_values, sparse_offsets, emb_table,
           dw0, db0, dw1, db1, dw2, db2,
           ow0, ob0, ow1, ob1, ow2, ob2, ow3, ob3, ow4, ob4):
  del sparse_offsets
  ow0a = ow0[:D]
  ow0b = ow0[D:]
  r = lambda b: b.reshape(1, -1)
  h = B // SPLIT
  nch = (h * F) // (NW * CHUNK)
  sv2 = sparse_values.reshape(B, F)
  outs = []
  for k in range(SPLIT):
    idxf = sv2[k * h:(k + 1) * h].T.reshape(NW, nch, CHUNK)
    gathered = _sc_gather(nch)(idxf, emb_table)
    g3 = gathered.reshape(F, h, D)
    outs.append(_dense_call(
        dense_features[k * h:(k + 1) * h], g3,
        dw0, r(db0), dw1, r(db1), dw2, r(db2),
        ow0a, ow0b, r(ob0), ow1, r(ob1), ow2, r(ob2), ow3, r(ob3),
        ow4, r(ob4)))
  return jnp.concatenate(outs, axis=0)


# speedup vs baseline: 14.4087x; 1.0640x over previous
import functools

import jax
import jax.numpy as jnp
import numpy as np
from jax import lax
from jax.experimental import pallas as pl
from jax.experimental.pallas import tpu as pltpu
from jax.experimental.pallas import tpu_sc as plsc

B = 16384
F = 13
D = 128
NF = F + 1

NC = 2
NS = 16
NW = NC * NS
CHUNK = 128
SPLIT = 2


def _make_gather_body(nch):
  per_w = nch * CHUNK

  def _gather_body(idx_hbm, table_hbm, out_hbm, idx_v, buf0, buf1, sem):
    cid = lax.axis_index("c")
    sid = lax.axis_index("s")
    wid = sid * NC + cid
    outbase = wid * per_w
    pltpu.sync_copy(idx_hbm.at[wid], idx_v)
    bufs = (buf0, buf1)
    pltpu.async_copy(table_hbm.at[idx_v.at[0]], bufs[0], sem.at[0])

    @pl.loop(0, nch, step=2)
    def _(c):
      for b in range(2):
        ci = c + b
        pltpu.make_async_copy(table_hbm.at[idx_v.at[ci]], bufs[b], sem.at[b]).wait()
        nxt = lax.rem(ci + 1, nch)
        pltpu.async_copy(table_hbm.at[idx_v.at[nxt]], bufs[1 - b], sem.at[1 - b])
        pltpu.sync_copy(bufs[b], out_hbm.at[pl.ds(outbase + ci * CHUNK, CHUNK)])

    pltpu.make_async_copy(table_hbm.at[idx_v.at[0]], bufs[0], sem.at[0]).wait()

  return _gather_body


@functools.cache
def _sc_gather(nch):
  return pl.kernel(
      _make_gather_body(nch),
      out_type=jax.ShapeDtypeStruct((NW * nch * CHUNK, D), jnp.float32),
      mesh=plsc.VectorSubcoreMesh(
          core_axis_name="c", subcore_axis_name="s", num_cores=NC,
          num_subcores=NS,
      ),
      scratch_types=[
          pltpu.VMEM((nch, CHUNK), jnp.int32),
          pltpu.VMEM((CHUNK, D), jnp.float32),
          pltpu.VMEM((CHUNK, D), jnp.float32),
          pltpu.SemaphoreType.DMA((2,)),
      ],
  )


BB = 512
NBLK = B // BB


TS = 16


def _dense_body(df_ref, g_ref, dw0, db0, dw1, db1, dw2, db2,
                ow0, ob0, ow1, ob1, ow2, ob2, ow3, ob3, ow4, ob4,
                out_ref, ed0, flat0, ed1, flat1):
  f32 = jnp.float32
  relu = lambda v: jnp.maximum(v, 0.0)
  dot = lambda a, b: jnp.dot(a, b, preferred_element_type=f32)

  def mlp(off):
    x = df_ref[off:off + BB, :]
    x = relu(dot(x, dw0[:]) + db0[:])
    x = relu(dot(x, dw1[:]) + db1[:])
    return relu(dot(x, dw2[:]) + db2[:])

  def int_group(ed, goff, flat_s, bts):
    for bt in bts:
      sl = pl.ds(goff + bt * TS, TS)
      fts = ([ed[bt * TS:(bt + 1) * TS]]
             + [g_ref[f, sl, :] for f in range(F)])
      cols = []
      for f in range(NF):
        for h in range(f + 1, NF):
          cols.append(jnp.sum(fts[f] * fts[h], axis=1, keepdims=True))
      flat_s[pl.ds(bt * TS, TS), :] = jnp.concatenate(cols, axis=1)

  G4 = BB // TS // 4

  def half(ed_s, flat_s, off, ed_new):
    y = relu(dot(ed_s[:], ow0[0:D, :]) + dot(flat_s[:], ow0[D:, :]) + ob0[:])
    int_group(ed_new, off, flat_s, range(0, G4))
    y = relu(dot(y, ow1[:]) + ob1[:])
    int_group(ed_new, off, flat_s, range(G4, 2 * G4))
    y = relu(dot(y, ow2[:]) + ob2[:])
    int_group(ed_new, off, flat_s, range(2 * G4, 3 * G4))
    y = relu(dot(y, ow3[:]) + ob3[:])
    int_group(ed_new, off, flat_s, range(3 * G4, 4 * G4))
    out_ref[off:off + BB, :] = dot(y, ow4[:]) + ob4[:]
    ed_s[:] = ed_new

  ed_a = mlp(0)
  ed_b = mlp(BB)
  half(ed0, flat0, 0, ed_a)
  half(ed1, flat1, BB, ed_b)


def _full(shape):
  return pl.BlockSpec(shape, lambda i: (0, 0))


def _dense_call(df, g2, dw0, db0, dw1, db1, dw2, db2,
                ow0, ob0, ow1, ob1, ow2, ob2, ow3, ob3, ow4, ob4):
  nb2 = df.shape[0] // (2 * BB)
  clamp = lambda i: (jnp.minimum(i, nb2 - 1), 0)
  lag = lambda i: (jnp.maximum(i - 1, 0), 0)
  in_specs = [
      pl.BlockSpec((2 * BB, 13), clamp),
      pl.BlockSpec((F, 2 * BB, D), lambda i: (0, jnp.minimum(i, nb2 - 1), 0)),
  ]
  for w in (dw0, db0, dw1, db1, dw2, db2,
            ow0, ob0, ow1, ob1, ow2, ob2, ow3, ob3, ow4, ob4):
    in_specs.append(_full(w.shape))
  return pl.pallas_call(
      _dense_body,
      grid=(nb2 + 1,),
      in_specs=in_specs,
      out_specs=pl.BlockSpec((2 * BB, 1), lag),
      out_shape=jax.ShapeDtypeStruct((df.shape[0], 1), jnp.float32),
      scratch_shapes=[
          pltpu.VMEM((BB, D), jnp.float32),
          pltpu.VMEM((BB, 91), jnp.float32),
          pltpu.VMEM((BB, D), jnp.float32),
          pltpu.VMEM((BB, 91), jnp.float32),
      ],
      compiler_params=pltpu.CompilerParams(
          dimension_semantics=("arbitrary",),
      ),
  )(df, g2, dw0, db0, dw1, db1, dw2, db2,
    ow0, ob0, ow1, ob1, ow2, ob2, ow3, ob3, ow4, ob4)


@jax.jit
def kernel(dense_features, sparse_values, sparse_offsets, emb_table,
           dw0, db0, dw1, db1, dw2, db2,
           ow0, ob0, ow1, ob1, ow2, ob2, ow3, ob3, ow4, ob4):
  del sparse_offsets
  r = lambda b: b.reshape(1, -1)
  h = B // SPLIT
  nch = (h * F) // (NW * CHUNK)
  sv2 = sparse_values.reshape(B, F)
  outs = []
  for k in range(SPLIT):
    idxf = sv2[k * h:(k + 1) * h].T.reshape(NW, nch, CHUNK)
    gathered = _sc_gather(nch)(idxf, emb_table)
    g3 = gathered.reshape(F, h, D)
    outs.append(_dense_call(
        dense_features[k * h:(k + 1) * h], g3,
        dw0, r(db0), dw1, r(db1), dw2, r(db2),
        ow0, r(ob0), ow1, r(ob1), ow2, r(ob2), ow3, r(ob3),
        ow4, r(ob4)))
  return jnp.concatenate(outs, axis=0)


# speedup vs baseline: 14.7351x; 1.0227x over previous
import functools

import jax
import jax.numpy as jnp
import numpy as np
from jax import lax
from jax.experimental import pallas as pl
from jax.experimental.pallas import tpu as pltpu
from jax.experimental.pallas import tpu_sc as plsc

B = 16384
F = 13
D = 128
NF = F + 1

NC = 2
NS = 16
NW = NC * NS
CHUNK = 104
SPLIT = 2
NBUF = 4
PRIME = 3


def _make_gather_body(nch):
  per_w = nch * CHUNK

  def _gather_body(idx_hbm, table_hbm, out_hbm, idx_v, b0, b1, b2, b3, sem):
    cid = lax.axis_index("c")
    sid = lax.axis_index("s")
    wid = sid * NC + cid
    outbase = wid * per_w
    pltpu.sync_copy(idx_hbm.at[wid], idx_v)
    bufs = (b0, b1, b2, b3)
    for p in range(PRIME):
      pltpu.async_copy(table_hbm.at[idx_v.at[p]], bufs[p], sem.at[p])

    @pl.loop(0, nch, step=NBUF)
    def _(c):
      for b in range(NBUF):
        ci = c + b
        pltpu.make_async_copy(table_hbm.at[idx_v.at[ci]], bufs[b], sem.at[b]).wait()
        nxt = lax.rem(ci + PRIME, nch)
        nb_ = (b + PRIME) % NBUF
        pltpu.async_copy(table_hbm.at[idx_v.at[nxt]], bufs[nb_], sem.at[nb_])
        pltpu.sync_copy(bufs[b], out_hbm.at[pl.ds(outbase + ci * CHUNK, CHUNK)])

    for p in range(PRIME):
      bd = (nch + p) % NBUF
      pltpu.make_async_copy(table_hbm.at[idx_v.at[p]], bufs[bd], sem.at[bd]).wait()

  return _gather_body


@functools.cache
def _sc_gather(nch):
  return pl.kernel(
      _make_gather_body(nch),
      out_type=jax.ShapeDtypeStruct((NW * nch * CHUNK, D), jnp.float32),
      mesh=plsc.VectorSubcoreMesh(
          core_axis_name="c", subcore_axis_name="s", num_cores=NC,
          num_subcores=NS,
      ),
      scratch_types=[
          pltpu.VMEM((nch, CHUNK), jnp.int32),
          pltpu.VMEM((CHUNK, D), jnp.float32),
          pltpu.VMEM((CHUNK, D), jnp.float32),
          pltpu.VMEM((CHUNK, D), jnp.float32),
          pltpu.VMEM((CHUNK, D), jnp.float32),
          pltpu.SemaphoreType.DMA((NBUF,)),
      ],
  )


BB = 512
NBLK = B // BB


TS = 16


def _dense_body(df_ref, g_ref, dw0, db0, dw1, db1, dw2, db2,
                ow0, ob0, ow1, ob1, ow2, ob2, ow3, ob3, ow4, ob4,
                out_ref, ed0, flat0, ed1, flat1):
  f32 = jnp.float32
  relu = lambda v: jnp.maximum(v, 0.0)
  dot = lambda a, b: jnp.dot(a, b, preferred_element_type=f32)

  def mlp(off):
    x = df_ref[off:off + BB, :]
    x = relu(dot(x, dw0[:]) + db0[:])
    x = relu(dot(x, dw1[:]) + db1[:])
    return relu(dot(x, dw2[:]) + db2[:])

  def int_group(ed, goff, flat_s, bts):
    for bt in bts:
      sl = pl.ds(goff + bt * TS, TS)
      fts = ([ed[bt * TS:(bt + 1) * TS]]
             + [g_ref[f, sl, :] for f in range(F)])
      cols = []
      for f in range(NF):
        for h in range(f + 1, NF):
          cols.append(jnp.sum(fts[f] * fts[h], axis=1, keepdims=True))
      flat_s[pl.ds(bt * TS, TS), :] = jnp.concatenate(cols, axis=1)

  NG = BB // TS // 4

  ed_a = mlp(0)
  ed_b = mlp(BB)
  y = relu(dot(ed0[:], ow0[0:D, :]) + dot(flat0[:], ow0[D:, :]) + ob0[:])
  z = relu(dot(ed1[:], ow0[0:D, :]) + dot(flat1[:], ow0[D:, :]) + ob0[:])
  int_group(ed_a, 0, flat0, range(0, NG))
  int_group(ed_b, BB, flat1, range(0, NG))
  y = relu(dot(y, ow1[:]) + ob1[:])
  z = relu(dot(z, ow1[:]) + ob1[:])
  int_group(ed_a, 0, flat0, range(NG, 2 * NG))
  int_group(ed_b, BB, flat1, range(NG, 2 * NG))
  y = relu(dot(y, ow2[:]) + ob2[:])
  z = relu(dot(z, ow2[:]) + ob2[:])
  int_group(ed_a, 0, flat0, range(2 * NG, 3 * NG))
  int_group(ed_b, BB, flat1, range(2 * NG, 3 * NG))
  y = relu(dot(y, ow3[:]) + ob3[:])
  z = relu(dot(z, ow3[:]) + ob3[:])
  int_group(ed_a, 0, flat0, range(3 * NG, 4 * NG))
  int_group(ed_b, BB, flat1, range(3 * NG, 4 * NG))
  out_ref[0:BB, :] = dot(y, ow4[:]) + ob4[:]
  out_ref[BB:2 * BB, :] = dot(z, ow4[:]) + ob4[:]
  ed0[:] = ed_a
  ed1[:] = ed_b


def _full(shape):
  return pl.BlockSpec(shape, lambda i: (0, 0))


def _dense_call(df, g2, dw0, db0, dw1, db1, dw2, db2,
                ow0, ob0, ow1, ob1, ow2, ob2, ow3, ob3, ow4, ob4):
  nb2 = df.shape[0] // (2 * BB)
  clamp = lambda i: (jnp.minimum(i, nb2 - 1), 0)
  lag = lambda i: (jnp.maximum(i - 1, 0), 0)
  in_specs = [
      pl.BlockSpec((2 * BB, 13), clamp),
      pl.BlockSpec((F, 2 * BB, D), lambda i: (0, jnp.minimum(i, nb2 - 1), 0)),
  ]
  for w in (dw0, db0, dw1, db1, dw2, db2,
            ow0, ob0, ow1, ob1, ow2, ob2, ow3, ob3, ow4, ob4):
    in_specs.append(_full(w.shape))
  return pl.pallas_call(
      _dense_body,
      grid=(nb2 + 1,),
      in_specs=in_specs,
      out_specs=pl.BlockSpec((2 * BB, 1), lag),
      out_shape=jax.ShapeDtypeStruct((df.shape[0], 1), jnp.float32),
      scratch_shapes=[
          pltpu.VMEM((BB, D), jnp.float32),
          pltpu.VMEM((BB, 91), jnp.float32),
          pltpu.VMEM((BB, D), jnp.float32),
          pltpu.VMEM((BB, 91), jnp.float32),
      ],
      compiler_params=pltpu.CompilerParams(
          dimension_semantics=("arbitrary",),
      ),
  )(df, g2, dw0, db0, dw1, db1, dw2, db2,
    ow0, ob0, ow1, ob1, ow2, ob2, ow3, ob3, ow4, ob4)


@jax.jit
def kernel(dense_features, sparse_values, sparse_offsets, emb_table,
           dw0, db0, dw1, db1, dw2, db2,
           ow0, ob0, ow1, ob1, ow2, ob2, ow3, ob3, ow4, ob4):
  del sparse_offsets
  r = lambda b: b.reshape(1, -1)
  h = B // SPLIT
  nch = (h * F) // (NW * CHUNK)
  sv2 = sparse_values.reshape(B, F)
  outs = []
  for k in range(SPLIT):
    idxf = sv2[k * h:(k + 1) * h].T.reshape(NW, nch, CHUNK)
    gathered = _sc_gather(nch)(idxf, emb_table)
    g3 = gathered.reshape(F, h, D)
    outs.append(_dense_call(
        dense_features[k * h:(k + 1) * h], g3,
        dw0, r(db0), dw1, r(db1), dw2, r(db2),
        ow0, r(ob0), ow1, r(ob1), ow2, r(ob2), ow3, r(ob3),
        ow4, r(ob4)))
  return jnp.concatenate(outs, axis=0)
